# Initial kernel scaffold; baseline (speedup 1.0000x reference)
import jax
import jax.numpy as jnp
from jax import lax
from jax.experimental import pallas as pl
from jax.experimental.pallas import tpu as pltpu
from jax.experimental.pallas import tpu_sc as plsc

N = 524288
HIDDEN = 64
MAX_COORD = 64
VOCAB = (MAX_COORD + 1) ** 3

NC = 2
NS = 16
NW = NC * NS
PER_W = N // NW
CHUNK = 1024
NCHUNK = PER_W // CHUNK
GCH = 128
NG = CHUNK // GCH


def _body(xcol, ycol, zcol, embed, out, xv, yv, zv, idxv, rows, sem):
  wid = lax.axis_index("s") * NC + lax.axis_index("c")
  base = wid * PER_W

  @pl.loop(0, NCHUNK)
  def _chunk(c):
    cbase = pl.multiple_of(base + c * CHUNK, CHUNK)
    pltpu.sync_copy(xcol.at[pl.ds(cbase, CHUNK)], xv)
    pltpu.sync_copy(ycol.at[pl.ds(cbase, CHUNK)], yv)
    pltpu.sync_copy(zcol.at[pl.ds(cbase, CHUNK)], zv)

    for g in range(NG):

      @pl.loop(0, GCH // 16)
      def _grp(i):
        s = pl.ds(g * GCH + i * 16, 16)
        x = jnp.clip(xv[s], 0, MAX_COORD)
        y = jnp.clip(yv[s], 0, MAX_COORD)
        z = jnp.clip(zv[s], 0, MAX_COORD)
        idxv[g, pl.ds(i * 16, 16)] = (
            x * ((MAX_COORD + 1) ** 2) + y * (MAX_COORD + 1) + z
        )

    copies = [
        pltpu.async_copy(
            embed.at[idxv.at[g]], rows.at[pl.ds(g * GCH, GCH)], sem
        )
        for g in range(NG)
    ]
    for cp in copies:
      cp.wait()

    pltpu.sync_copy(rows, out.at[pl.ds(cbase, CHUNK), :])


@jax.jit
def kernel(coords, embed):
  coords = coords.astype(jnp.int32)
  xcol = coords[:, 1]
  ycol = coords[:, 2]
  zcol = coords[:, 3]

  mesh = plsc.VectorSubcoreMesh(core_axis_name="c", subcore_axis_name="s")
  f = pl.kernel(
      _body,
      out_type=jax.ShapeDtypeStruct((N, HIDDEN), jnp.float32),
      mesh=mesh,
      scratch_types=[
          pltpu.VMEM((CHUNK,), jnp.int32),
          pltpu.VMEM((CHUNK,), jnp.int32),
          pltpu.VMEM((CHUNK,), jnp.int32),
          pltpu.VMEM((NG, GCH), jnp.int32),
          pltpu.VMEM((CHUNK, HIDDEN), jnp.float32),
          pltpu.SemaphoreType.DMA,
      ],
  )
  return f(xcol, ycol, zcol, embed)


# speedup vs baseline: 1.0378x; 1.0378x over previous
import jax
import jax.numpy as jnp
from jax import lax
from jax.experimental import pallas as pl
from jax.experimental.pallas import tpu as pltpu
from jax.experimental.pallas import tpu_sc as plsc

N = 524288
HIDDEN = 64
MAX_COORD = 64
VOCAB = (MAX_COORD + 1) ** 3

NC = 2
NS = 16
NW = NC * NS
PER_W = N // NW
CHUNK = 1024
NCHUNK = PER_W // CHUNK
GCH = 128
NG = CHUNK // GCH


def _body(xcol, ycol, zcol, embed, out, xv, yv, zv, idxv, rows, sem):
  wid = lax.axis_index("s") * NC + lax.axis_index("c")
  base = wid * PER_W

  @pl.loop(0, NCHUNK)
  def _chunk(c):
    cbase = pl.multiple_of(base + c * CHUNK, CHUNK)
    pltpu.sync_copy(xcol.at[pl.ds(cbase, CHUNK)], xv)
    pltpu.sync_copy(ycol.at[pl.ds(cbase, CHUNK)], yv)
    pltpu.sync_copy(zcol.at[pl.ds(cbase, CHUNK)], zv)

    for g in range(NG):

      @pl.loop(0, GCH // 16)
      def _grp(i):
        s = pl.ds(g * GCH + i * 16, 16)
        x = jnp.clip(xv[s], 0, MAX_COORD)
        y = jnp.clip(yv[s], 0, MAX_COORD)
        z = jnp.clip(zv[s], 0, MAX_COORD)
        idxv[g, pl.ds(i * 16, 16)] = (
            x * ((MAX_COORD + 1) ** 2) + y * (MAX_COORD + 1) + z
        )

    copies = [
        pltpu.async_copy(
            embed.at[idxv.at[g]], rows.at[pl.ds(g * GCH, GCH)], sem
        )
        for g in range(NG)
    ]
    for cp in copies:
      cp.wait()

    pltpu.sync_copy(rows, out.at[pl.ds(cbase, CHUNK), :])


@jax.jit
def kernel(coords, embed):
  coords = coords.astype(jnp.int32)
  xcol = coords[:, 1]
  ycol = coords[:, 2]
  zcol = coords[:, 3]

  mesh = plsc.VectorSubcoreMesh(core_axis_name="c", subcore_axis_name="s")
  f = pl.kernel(
      _body,
      out_type=jax.ShapeDtypeStruct((N, HIDDEN), jnp.float32),
      mesh=mesh,
      scratch_types=[
          pltpu.VMEM((CHUNK,), jnp.int32),
          pltpu.VMEM((CHUNK,), jnp.int32),
          pltpu.VMEM((CHUNK,), jnp.int32),
          pltpu.VMEM((NG, GCH), jnp.int32),
          pltpu.VMEM((CHUNK, HIDDEN), jnp.float32),
          pltpu.SemaphoreType.DMA,
      ],
      compiler_params=pltpu.CompilerParams(use_tc_tiling_on_sc=False),
  )
  return f(xcol, ycol, zcol, embed)


# speedup vs baseline: 1.0740x; 1.0349x over previous
import jax
import jax.numpy as jnp
from jax import lax
from jax.experimental import pallas as pl
from jax.experimental.pallas import tpu as pltpu
from jax.experimental.pallas import tpu_sc as plsc

N = 524288
HIDDEN = 64
MAX_COORD = 64
VOCAB = (MAX_COORD + 1) ** 3

NC = 2
NS = 16
NW = NC * NS
PER_W = N // NW
CHUNK = 512
NCHUNK = PER_W // CHUNK
GCH = 128
NG = CHUNK // GCH


def _body(cols, embed, out, cv0, cv1, idx0, idx1, rows0, rows1,
          csem0, csem1, gsem, osem0, osem1):
  cvs = (cv0, cv1)
  idxs = (idx0, idx1)
  rowss = (rows0, rows1)
  csems = (csem0, csem1)
  osems = (osem0, osem1)

  wid = lax.axis_index("s") * NC + lax.axis_index("c")
  base = wid * PER_W

  def coords_copy(c, b):
    cbase = base + c * CHUNK
    return pltpu.make_async_copy(
        cols.at[:, pl.ds(cbase, CHUNK)], cvs[b], csems[b]
    )

  def process(c, b, prefetch):
    cbase = base + c * CHUNK
    out_copy = pltpu.make_async_copy(
        rowss[b], out.at[pl.ds(cbase, CHUNK), :], osems[b]
    )

    coords_copy(c, b).wait()

    cv = cvs[b]
    idxv = idxs[b]
    for g in range(NG):

      @pl.loop(0, GCH // 16)
      def _grp(k):
        s = pl.ds(g * GCH + k * 16, 16)
        x = jnp.clip(cv[0, s], 0, MAX_COORD)
        y = jnp.clip(cv[1, s], 0, MAX_COORD)
        z = jnp.clip(cv[2, s], 0, MAX_COORD)
        idxv[g, pl.ds(k * 16, 16)] = (
            x * ((MAX_COORD + 1) ** 2) + y * (MAX_COORD + 1) + z
        )

    if prefetch:
      coords_copy(c + 2, b).start()

    if prefetch:

      @pl.when(c >= 2)
      def _():
        out_copy.wait()

    else:
      out_copy.wait()

    copies = [
        pltpu.async_copy(
            embed.at[idxv.at[g]], rowss[b].at[pl.ds(g * GCH, GCH)], gsem
        )
        for g in range(NG)
    ]
    for cp in copies:
      cp.wait()

    out_copy.start()

  coords_copy(0, 0).start()
  coords_copy(1, 1).start()

  @pl.loop(0, NCHUNK // 2 - 1)
  def _iter(i):
    for b in range(2):
      process(i * 2 + b, b, prefetch=True)

  process(NCHUNK - 2, 0, prefetch=False)
  process(NCHUNK - 1, 1, prefetch=False)

  for b, c in ((0, NCHUNK - 2), (1, NCHUNK - 1)):
    cbase = base + c * CHUNK
    pltpu.make_async_copy(
        rowss[b], out.at[pl.ds(cbase, CHUNK), :], osems[b]
    ).wait()


@jax.jit
def kernel(coords, embed):
  coords = coords.astype(jnp.int32)
  cols = jnp.stack([coords[:, 1], coords[:, 2], coords[:, 3]])

  mesh = plsc.VectorSubcoreMesh(core_axis_name="c", subcore_axis_name="s")
  f = pl.kernel(
      _body,
      out_type=jax.ShapeDtypeStruct((N, HIDDEN), jnp.float32),
      mesh=mesh,
      scratch_types=[
          pltpu.VMEM((3, CHUNK), jnp.int32),
          pltpu.VMEM((3, CHUNK), jnp.int32),
          pltpu.VMEM((NG, GCH), jnp.int32),
          pltpu.VMEM((NG, GCH), jnp.int32),
          pltpu.VMEM((CHUNK, HIDDEN), jnp.float32),
          pltpu.VMEM((CHUNK, HIDDEN), jnp.float32),
          pltpu.SemaphoreType.DMA,
          pltpu.SemaphoreType.DMA,
          pltpu.SemaphoreType.DMA,
          pltpu.SemaphoreType.DMA,
          pltpu.SemaphoreType.DMA,
      ],
      compiler_params=pltpu.CompilerParams(use_tc_tiling_on_sc=False),
  )
  return f(cols, embed)


# speedup vs baseline: 1.2797x; 1.1915x over previous
import jax
import jax.numpy as jnp
from jax import lax
from jax.experimental import pallas as pl
from jax.experimental.pallas import tpu as pltpu
from jax.experimental.pallas import tpu_sc as plsc

N = 524288
HIDDEN = 64
MAX_COORD = 64
VOCAB = (MAX_COORD + 1) ** 3
VPAD = VOCAB + 7

NC = 2
NS = 16
NW = NC * NS
PER_W = N // NW
CHUNK = 256
NCHUNK = PER_W // CHUNK
GCH = 128
NG = CHUNK // GCH


def _body(xcol, ycol, zcol, embed, out, cv0, cv1, idx0, idx1, rows0, rows1,
          csem0, csem1, gsem, osem0, osem1):
  cvs = (cv0, cv1)
  idxs = (idx0, idx1)
  rowss = (rows0, rows1)
  csems = (csem0, csem1)
  osems = (osem0, osem1)
  cols = (xcol, ycol, zcol)

  wid = lax.axis_index("s") * NC + lax.axis_index("c")
  base = wid * PER_W

  def coords_copies(c, b):
    cbase = base + c * CHUNK
    return [
        pltpu.make_async_copy(
            cols[j].at[pl.ds(cbase, CHUNK)], cvs[b].at[j], csems[b]
        )
        for j in range(3)
    ]

  def process(c, b, prefetch):
    cbase = base + c * CHUNK
    out_copy = pltpu.make_async_copy(
        rowss[b].at[:, pl.ds(0, HIDDEN)],
        out.at[pl.ds(cbase, CHUNK), pl.ds(0, HIDDEN)],
        osems[b],
    )

    for cp in coords_copies(c, b):
      cp.wait()

    cv = cvs[b]
    idxv = idxs[b]
    for g in range(NG):

      @pl.loop(0, GCH // 16)
      def _grp(k):
        s = pl.ds(g * GCH + k * 16, 16)
        x = jnp.clip(cv[0, s], 0, MAX_COORD)
        y = jnp.clip(cv[1, s], 0, MAX_COORD)
        z = jnp.clip(cv[2, s], 0, MAX_COORD)
        idxv[g, pl.ds(k * 16, 16)] = (
            x * ((MAX_COORD + 1) ** 2) + y * (MAX_COORD + 1) + z
        )

    if prefetch:
      for cp in coords_copies(c + 2, b):
        cp.start()

    if prefetch:

      @pl.when(c >= 2)
      def _():
        out_copy.wait()

    else:
      out_copy.wait()

    copies = [
        pltpu.async_copy(
            embed.at[idxs[b].at[g]], rowss[b].at[pl.ds(g * GCH, GCH)], gsem
        )
        for g in range(NG)
    ]
    for cp in copies:
      cp.wait()

    out_copy.start()

  for cp in coords_copies(0, 0):
    cp.start()
  for cp in coords_copies(1, 1):
    cp.start()

  @pl.loop(0, NCHUNK // 2 - 1)
  def _iter(i):
    for b in range(2):
      process(i * 2 + b, b, prefetch=True)

  process(NCHUNK - 2, 0, prefetch=False)
  process(NCHUNK - 1, 1, prefetch=False)

  for b, c in ((0, NCHUNK - 2), (1, NCHUNK - 1)):
    cbase = base + c * CHUNK
    pltpu.make_async_copy(
        rowss[b].at[:, pl.ds(0, HIDDEN)],
        out.at[pl.ds(cbase, CHUNK), pl.ds(0, HIDDEN)],
        osems[b],
    ).wait()


@jax.jit
def kernel(coords, embed):
  coords = coords.astype(jnp.int32)
  xcol = coords[:, 1]
  ycol = coords[:, 2]
  zcol = coords[:, 3]
  embedp = jnp.pad(embed, ((0, VPAD - VOCAB), (0, 128 - HIDDEN)))

  mesh = plsc.VectorSubcoreMesh(core_axis_name="c", subcore_axis_name="s")
  f = pl.kernel(
      _body,
      out_type=jax.ShapeDtypeStruct((N, 128), jnp.float32),
      mesh=mesh,
      scratch_types=[
          pltpu.VMEM((3, CHUNK), jnp.int32),
          pltpu.VMEM((3, CHUNK), jnp.int32),
          pltpu.VMEM((NG, GCH), jnp.int32),
          pltpu.VMEM((NG, GCH), jnp.int32),
          pltpu.VMEM((CHUNK, 128), jnp.float32),
          pltpu.VMEM((CHUNK, 128), jnp.float32),
          pltpu.SemaphoreType.DMA,
          pltpu.SemaphoreType.DMA,
          pltpu.SemaphoreType.DMA,
          pltpu.SemaphoreType.DMA,
          pltpu.SemaphoreType.DMA,
      ],
      compiler_params=pltpu.CompilerParams(use_tc_tiling_on_sc=False),
  )
  out128 = f(xcol, ycol, zcol, embedp)
  return out128[:, :HIDDEN]


# speedup vs baseline: 1.3640x; 1.0658x over previous
import jax
import jax.numpy as jnp
from jax import lax
from jax.experimental import pallas as pl
from jax.experimental.pallas import tpu as pltpu
from jax.experimental.pallas import tpu_sc as plsc

N = 524288
HIDDEN = 64
MAX_COORD = 64
VOCAB = (MAX_COORD + 1) ** 3
VPAD = VOCAB + 7

NC = 2
NS = 16
NW = NC * NS
PER_W = N // NW
CHUNK = 256
NCHUNK = PER_W // CHUNK
GCH = 128
NG = CHUNK // GCH


def _body(xcol, ycol, zcol, embed, out, cv0, cv1, idx0, idx1, rows0, rows1,
          csem0, csem1, gsem, osem0, osem1):
  cvs = (cv0, cv1)
  idxs = (idx0, idx1)
  rowss = (rows0, rows1)
  csems = (csem0, csem1)
  osems = (osem0, osem1)
  cols = (xcol, ycol, zcol)

  wid = lax.axis_index("s") * NC + lax.axis_index("c")
  base = wid * PER_W

  def coords_copies(c, b):
    cbase = base + c * CHUNK
    return [
        pltpu.make_async_copy(
            cols[j].at[pl.ds(cbase, CHUNK)], cvs[b].at[j], csems[b]
        )
        for j in range(3)
    ]

  def out_copy(c, b):
    cbase = base + c * CHUNK
    return pltpu.make_async_copy(
        rowss[b],
        out.at[pl.ds(cbase, CHUNK), :],
        osems[b],
    )

  def process(c, b, prefetch):
    oc = out_copy(c, b)

    for cp in coords_copies(c, b):
      cp.wait()

    cv = cvs[b]
    idxv = idxs[b]
    for g in range(NG):

      @pl.loop(0, GCH // 16)
      def _grp(k):
        s = pl.ds(g * GCH + k * 16, 16)
        x = jnp.clip(cv[0, s], 0, MAX_COORD)
        y = jnp.clip(cv[1, s], 0, MAX_COORD)
        z = jnp.clip(cv[2, s], 0, MAX_COORD)
        idxv[g, pl.ds(k * 16, 16)] = (
            x * ((MAX_COORD + 1) ** 2) + y * (MAX_COORD + 1) + z
        )

    if prefetch:
      for cp in coords_copies(c + 2, b):
        cp.start()

    if prefetch:

      @pl.when(c >= 2)
      def _():
        oc.wait()

    else:
      oc.wait()

    copies = [
        pltpu.async_copy(
            embed.at[idxs[b].at[g]],
            rowss[b].at[pl.ds(g * GCH, GCH)],
            gsem,
        )
        for g in range(NG)
    ]
    for cp in copies:
      cp.wait()

    oc.start()

  for cp in coords_copies(0, 0):
    cp.start()
  for cp in coords_copies(1, 1):
    cp.start()

  @pl.loop(0, NCHUNK // 2 - 1)
  def _iter(i):
    for b in range(2):
      process(i * 2 + b, b, prefetch=True)

  process(NCHUNK - 2, 0, prefetch=False)
  process(NCHUNK - 1, 1, prefetch=False)

  for b, c in ((0, NCHUNK - 2), (1, NCHUNK - 1)):
    out_copy(c, b).wait()


@jax.jit
def kernel(coords, embed):
  coords = coords.astype(jnp.int32)
  xcol = coords[:, 1]
  ycol = coords[:, 2]
  zcol = coords[:, 3]
  embedp = jnp.pad(embed, ((0, VPAD - VOCAB), (0, 128 - HIDDEN)))

  mesh = plsc.VectorSubcoreMesh(core_axis_name="c", subcore_axis_name="s")
  f = pl.kernel(
      _body,
      out_type=jax.ShapeDtypeStruct((N, 128), jnp.float32),
      mesh=mesh,
      scratch_types=[
          pltpu.VMEM((3, CHUNK), jnp.int32),
          pltpu.VMEM((3, CHUNK), jnp.int32),
          pltpu.VMEM((NG, GCH), jnp.int32),
          pltpu.VMEM((NG, GCH), jnp.int32),
          pltpu.VMEM((CHUNK, 128), jnp.float32),
          pltpu.VMEM((CHUNK, 128), jnp.float32),
          pltpu.SemaphoreType.DMA,
          pltpu.SemaphoreType.DMA,
          pltpu.SemaphoreType.DMA,
          pltpu.SemaphoreType.DMA,
          pltpu.SemaphoreType.DMA,
      ],
      compiler_params=pltpu.CompilerParams(use_tc_tiling_on_sc=False),
  )
  out128 = f(xcol, ycol, zcol, embedp)
  return out128[:, :HIDDEN]


# speedup vs baseline: 1.3837x; 1.0145x over previous
import jax
import jax.numpy as jnp
from jax import lax
from jax.experimental import pallas as pl
from jax.experimental.pallas import tpu as pltpu
from jax.experimental.pallas import tpu_sc as plsc

N = 524288
HIDDEN = 64
MAX_COORD = 64
VOCAB = (MAX_COORD + 1) ** 3
VPAD = VOCAB + 7

NC = 2
NS = 16
NW = NC * NS
PER_W = N // NW
CHUNK = 256
NCHUNK = PER_W // CHUNK
GCH = 128
NG = CHUNK // GCH


def _body(xcol, ycol, zcol, embed, out, cv0, cv1, idx0, idx1, rows0, rows1,
          csem0, csem1, gsem0, gsem1, osem0, osem1):
  cvs = (cv0, cv1)
  idxs = (idx0, idx1)
  rowss = (rows0, rows1)
  csems = (csem0, csem1)
  gsems = (gsem0, gsem1)
  osems = (osem0, osem1)
  cols = (xcol, ycol, zcol)

  wid = lax.axis_index("s") * NC + lax.axis_index("c")
  base = wid * PER_W

  def coords_copies(c, b):
    cbase = base + c * CHUNK
    return [
        pltpu.make_async_copy(
            cols[j].at[pl.ds(cbase, CHUNK)], cvs[b].at[j], csems[b]
        )
        for j in range(3)
    ]

  def out_copy(c, b):
    cbase = base + c * CHUNK
    return pltpu.make_async_copy(
        rowss[b],
        out.at[pl.ds(cbase, CHUNK), :],
        osems[b],
    )

  def gathers(b):
    return [
        pltpu.make_async_copy(
            embed.at[idxs[b].at[g]],
            rowss[b].at[pl.ds(g * GCH, GCH)],
            gsems[b],
        )
        for g in range(NG)
    ]

  def fire(c, b, guarded):
    for cp in coords_copies(c, b):
      cp.wait()

    cv = cvs[b]
    idxv = idxs[b]
    for g in range(NG):

      @pl.loop(0, GCH // 16)
      def _grp(k):
        s = pl.ds(g * GCH + k * 16, 16)
        x = jnp.clip(cv[0, s], 0, MAX_COORD)
        y = jnp.clip(cv[1, s], 0, MAX_COORD)
        z = jnp.clip(cv[2, s], 0, MAX_COORD)
        idxv[g, pl.ds(k * 16, 16)] = (
            x * ((MAX_COORD + 1) ** 2) + y * (MAX_COORD + 1) + z
        )

    if guarded:

      @pl.when(c < NCHUNK - 2)
      def _():
        for cp in coords_copies(c + 2, b):
          cp.start()

      @pl.when(c >= 2)
      def _():
        out_copy(c, b).wait()

    else:
      for cp in coords_copies(c + 2, b):
        cp.start()

    for cp in gathers(b):
      cp.start()

  def drain(c, b):
    for cp in gathers(b):
      cp.wait()
    out_copy(c, b).start()

  for cp in coords_copies(0, 0):
    cp.start()
  for cp in coords_copies(1, 1):
    cp.start()
  fire(0, 0, guarded=False)

  @pl.loop(1, NCHUNK)
  def _iter(c):
    b = c % 2

    @pl.when(b == 0)
    def _():
      fire(c, 0, guarded=True)
      drain(c - 1, 1)

    @pl.when(b == 1)
    def _():
      fire(c, 1, guarded=True)
      drain(c - 1, 0)

  drain(NCHUNK - 1, 1)

  for b, c in ((0, NCHUNK - 2), (1, NCHUNK - 1)):
    out_copy(c, b).wait()


@jax.jit
def kernel(coords, embed):
  coords = coords.astype(jnp.int32)
  xcol = coords[:, 1]
  ycol = coords[:, 2]
  zcol = coords[:, 3]
  embedp = jnp.pad(embed, ((0, VPAD - VOCAB), (0, 128 - HIDDEN)))

  mesh = plsc.VectorSubcoreMesh(core_axis_name="c", subcore_axis_name="s")
  f = pl.kernel(
      _body,
      out_type=jax.ShapeDtypeStruct((N, 128), jnp.float32),
      mesh=mesh,
      scratch_types=[
          pltpu.VMEM((3, CHUNK), jnp.int32),
          pltpu.VMEM((3, CHUNK), jnp.int32),
          pltpu.VMEM((NG, GCH), jnp.int32),
          pltpu.VMEM((NG, GCH), jnp.int32),
          pltpu.VMEM((CHUNK, 128), jnp.float32),
          pltpu.VMEM((CHUNK, 128), jnp.float32),
          pltpu.SemaphoreType.DMA,
          pltpu.SemaphoreType.DMA,
          pltpu.SemaphoreType.DMA,
          pltpu.SemaphoreType.DMA,
          pltpu.SemaphoreType.DMA,
          pltpu.SemaphoreType.DMA,
      ],
      compiler_params=pltpu.CompilerParams(use_tc_tiling_on_sc=False),
  )
  out128 = f(xcol, ycol, zcol, embedp)
  return out128[:, :HIDDEN]
